# baseline (speedup 1.0000x reference)
import jax
import jax.numpy as jnp
from jax.experimental import pallas as pl
from jax.experimental.pallas import tpu as pltpu


def _make_body(C, N, F, W, P, OW, TAIL, KMAIN):
    def body(w_ref, b_ref, x_ref, xt_ref, atail_ref, adj_hbm, out_ref,
             acc_ref, pbuf, sem):
        i = pl.program_id(0)

        def copy(panel, slot):
            return pltpu.make_async_copy(
                adj_hbm.at[:, pl.ds(panel * W, W)],
                pbuf.at[slot],
                sem.at[slot],
            )

        @pl.when(i == 0)
        def _():
            copy(0, 0).start()

        @pl.when(i + 1 < P)
        def _():
            copy(i + 1, (i + 1) % 2).start()

        slot = i % 2

        @pl.when(i < P)
        def _():
            s_p = jnp.dot(x_ref[:, :].T, w_ref[:, :],
                          preferred_element_type=jnp.float32)
            copy(i, slot).wait()

            @pl.when(i == 0)
            def _():
                tail_s = jnp.dot(xt_ref[:, :TAIL].T, w_ref[:, :],
                                 preferred_element_type=jnp.float32)
                acc_ref[:N, :] = (
                    jnp.dot(pbuf[0], s_p,
                            preferred_element_type=jnp.float32)
                    + jnp.dot(atail_ref[:, :TAIL], tail_s,
                              preferred_element_type=jnp.float32)
                )

            @pl.when(i > 0)
            def _():
                acc_ref[:N, :] += jnp.dot(
                    pbuf[slot], s_p, preferred_element_type=jnp.float32)

        @pl.when(i >= P)
        def _():
            j = i - P
            chunk = acc_ref[pl.ds(j * OW, OW), :]
            out_ref[:, :] = chunk.T + b_ref[:, :]

    return body


def kernel(input, adj, weight, bias):
    C, N = input.shape
    F = weight.shape[1]

    W = 384
    P = N // W
    KMAIN = P * W
    TAIL = N - KMAIN
    LT = KMAIN // 128
    OW = 1024
    NW = pl.cdiv(N, OW)
    G = P + NW

    bias_col = bias.reshape(F, 1)

    def x_map(i):
        return (0, jnp.minimum(i, P - 1))

    def out_map(i):
        return (0, jnp.maximum(i - P, 0))

    out = pl.pallas_call(
        _make_body(C, N, F, W, P, OW, TAIL, KMAIN),
        grid=(G,),
        in_specs=[
            pl.BlockSpec((C, F), lambda i: (0, 0)),
            pl.BlockSpec((F, 1), lambda i: (0, 0)),
            pl.BlockSpec((C, W), x_map),
            pl.BlockSpec((C, 128), lambda i: (0, LT)),
            pl.BlockSpec((N, 128), lambda i: (0, LT)),
            pl.BlockSpec(memory_space=pl.ANY),
        ],
        out_specs=pl.BlockSpec((F, OW), out_map),
        out_shape=jax.ShapeDtypeStruct((F, N), jnp.float32),
        scratch_shapes=[
            pltpu.VMEM((NW * OW, F), jnp.float32),
            pltpu.VMEM((2, N, W), jnp.float32),
            pltpu.SemaphoreType.DMA((2,)),
        ],
    )(weight, bias_col, input, input, adj, adj)
    return out


# speedup vs baseline: 1.0074x; 1.0074x over previous
import jax
import jax.numpy as jnp
from jax.experimental import pallas as pl
from jax.experimental.pallas import tpu as pltpu


def _make_body(C, N, F, W, P, OW, TAIL, KMAIN):
    def body(w_ref, b_ref, x_ref, xt_ref, atail_ref, adj_hbm, out_ref,
             acc_ref, pbuf, sem):
        i = pl.program_id(0)

        def copy(panel, slot):
            return pltpu.make_async_copy(
                adj_hbm.at[:, pl.ds(panel * W, W)],
                pbuf.at[slot],
                sem.at[slot],
            )

        @pl.when(i == 0)
        def _():
            copy(0, 0).start()

        @pl.when(i + 1 < P)
        def _():
            copy(i + 1, (i + 1) % 2).start()

        slot = i % 2

        @pl.when(i < P)
        def _():
            s_p = jnp.dot(x_ref[:, :].T, w_ref[:, :],
                          preferred_element_type=jnp.float32)
            copy(i, slot).wait()

            @pl.when(i == 0)
            def _():
                tail_s = jnp.dot(xt_ref[:, :TAIL].T, w_ref[:, :],
                                 preferred_element_type=jnp.float32)
                acc_ref[:N, :] = (
                    jnp.dot(pbuf[0], s_p,
                            preferred_element_type=jnp.float32)
                    + jnp.dot(atail_ref[:, :TAIL], tail_s,
                              preferred_element_type=jnp.float32)
                )

            @pl.when(i > 0)
            def _():
                acc_ref[:N, :] += jnp.dot(
                    pbuf[slot], s_p, preferred_element_type=jnp.float32)

        @pl.when(i >= P)
        def _():
            j = i - P
            chunk = acc_ref[pl.ds(j * OW, OW), :]
            out_ref[:, :] = chunk.T + b_ref[:, :]

    return body


def kernel(input, adj, weight, bias):
    C, N = input.shape
    F = weight.shape[1]

    W = 256
    P = N // W
    KMAIN = P * W
    TAIL = N - KMAIN
    LT = KMAIN // 128
    OW = 1024
    NW = pl.cdiv(N, OW)
    G = P + NW

    bias_col = bias.reshape(F, 1)

    def x_map(i):
        return (0, jnp.minimum(i, P - 1))

    def out_map(i):
        return (0, jnp.maximum(i - P, 0))

    out = pl.pallas_call(
        _make_body(C, N, F, W, P, OW, TAIL, KMAIN),
        grid=(G,),
        in_specs=[
            pl.BlockSpec((C, F), lambda i: (0, 0)),
            pl.BlockSpec((F, 1), lambda i: (0, 0)),
            pl.BlockSpec((C, W), x_map),
            pl.BlockSpec((C, 128), lambda i: (0, LT)),
            pl.BlockSpec((N, 128), lambda i: (0, LT)),
            pl.BlockSpec(memory_space=pl.ANY),
        ],
        out_specs=pl.BlockSpec((F, OW), out_map),
        out_shape=jax.ShapeDtypeStruct((F, N), jnp.float32),
        scratch_shapes=[
            pltpu.VMEM((NW * OW, F), jnp.float32),
            pltpu.VMEM((2, N, W), jnp.float32),
            pltpu.SemaphoreType.DMA((2,)),
        ],
    )(weight, bias_col, input, input, adj, adj)
    return out


# speedup vs baseline: 1.0577x; 1.0499x over previous
import jax
import jax.numpy as jnp
from jax.experimental import pallas as pl
from jax.experimental.pallas import tpu as pltpu


def _make_fused(N, TN, SUB, G):
    NSUB = TN // SUB
    REM = N - (G - 1) * TN

    def _fused(x_ref, w_ref, b_ref, adj_hbm, out_ref, s_ref, abuf, sem):
        i = pl.program_id(0)

        def full_copies(block, slot):
            return [
                pltpu.make_async_copy(
                    adj_hbm.at[pl.ds(block * TN + k * SUB, SUB), :],
                    abuf.at[slot, pl.ds(k * SUB, SUB), :],
                    sem.at[slot],
                )
                for k in range(NSUB)
            ]

        def tail_copy(slot):
            return pltpu.make_async_copy(
                adj_hbm.at[pl.ds((G - 1) * TN, REM), :],
                abuf.at[slot, pl.ds(0, REM), :],
                sem.at[slot],
            )

        def issue(block, slot):
            @pl.when(block < G - 1)
            def _():
                for c in full_copies(block, slot):
                    c.start()

            @pl.when(block == G - 1)
            def _():
                tail_copy(slot).start()

        def wait(block, slot):
            @pl.when(block < G - 1)
            def _():
                for c in full_copies(block, slot):
                    c.wait()

            @pl.when(block == G - 1)
            def _():
                tail_copy(slot).wait()

        @pl.when(i == 0)
        def _():
            issue(0, 0)
            xt = x_ref[:, :].astype(jnp.bfloat16).T
            w = w_ref[:, :].astype(jnp.bfloat16)
            s = jnp.dot(xt, w, preferred_element_type=jnp.float32)
            s_ref[:, :] = s

        @pl.when(i + 1 < G)
        def _():
            issue(i + 1, (i + 1) % 2)

        wait(i, i % 2)

        slot = i % 2
        a = abuf[slot]
        acc = jnp.dot(a, s_ref[:, :], preferred_element_type=jnp.float32)
        acc = acc + b_ref[:, :]
        out_ref[:, :] = acc.T

    return _fused


def kernel(input, adj, weight, bias):
    C, N = input.shape
    F = weight.shape[1]

    TN = 256
    SUB = 32
    G = pl.cdiv(N, TN)
    bias2 = bias.reshape(1, F)

    out = pl.pallas_call(
        _make_fused(N, TN, SUB, G),
        grid=(G,),
        in_specs=[
            pl.BlockSpec((C, N), lambda i: (0, 0)),
            pl.BlockSpec((C, F), lambda i: (0, 0)),
            pl.BlockSpec((1, F), lambda i: (0, 0)),
            pl.BlockSpec(memory_space=pl.ANY),
        ],
        out_specs=pl.BlockSpec((F, TN), lambda i: (0, i)),
        out_shape=jax.ShapeDtypeStruct((F, N), jnp.float32),
        scratch_shapes=[
            pltpu.VMEM((N, F), jnp.float32),
            pltpu.VMEM((2, TN, N), jnp.float32),
            pltpu.SemaphoreType.DMA((2,)),
        ],
    )(input, weight, bias2, adj)
    return out
